# baseline (speedup 1.0000x reference)
import functools

import jax
import jax.numpy as jnp
from jax import lax
from jax.experimental import pallas as pl
from jax.experimental.pallas import tpu as pltpu
from jax.experimental.pallas import tpu_sc as plsc

NB = 1024
NB2 = 2 * NB
L = 16
NC = 2
NS = 16
NW = NC * NS
NUM_C = 21
PIX = 512 * 512
CHUNK = PIX // NW
NBATCH = 4
CR = 64


def _stage1a_body(p_hbm, out_hbm, p_v, hist_v, in_sem):
    cid = lax.axis_index("c")
    sid = lax.axis_index("s")
    wid = sid * NC + cid
    base = wid * CHUNK

    ones = jnp.ones((L,), jnp.float32)
    zvec = jnp.zeros((L,), jnp.float32)
    nbf = jnp.float32(NB)
    nbm1 = jnp.full((L,), NB - 1, jnp.int32)

    def start_fetch(c, par):
        pltpu.make_async_copy(
            p_hbm.at[pl.ds(0, NBATCH), c, pl.ds(base, CHUNK)],
            p_v.at[par], in_sem).start()

    def wait_fetch(par):
        pltpu.make_async_copy(
            p_hbm.at[pl.ds(0, NBATCH), 0, pl.ds(base, CHUNK)],
            p_v.at[par], in_sem).wait()

    start_fetch(0, 0)

    def class_body(c, carry):
        par = lax.rem(c, 2)
        def zero_chunk(k, _):
            hist_v[pl.ds(k * L, L)] = zvec
            return 0
        lax.fori_loop(0, NB // L, zero_chunk, 0)

        wait_fetch(par)

        @pl.when(c + 1 < NUM_C)
        def _prefetch():
            start_fetch(c + 1, 1 - par)

        @plsc.parallel_loop(0, NBATCH * (CHUNK // L), unroll=8)
        def _vec(i):
            b = i // (CHUNK // L)
            j = i - b * (CHUNK // L)
            p = p_v[par, b, pl.ds(j * L, L)]
            bkt = jnp.minimum((p * nbf).astype(jnp.int32), nbm1)
            plsc.addupdate_scatter(hist_v, [bkt], ones)

        pltpu.sync_copy(hist_v, out_hbm.at[wid, c])
        return carry
    lax.fori_loop(0, NUM_C, class_body, 0)


def _plab_tc_body(p_ref, lab_ref, out_ref):
    p = p_ref[0]
    labv = lab_ref[0]
    cls = lax.broadcasted_iota(jnp.int32, (NUM_C, CR, 128), 0)
    onehot = (cls == labv[None]).astype(jnp.float32)
    out_ref[...] = jnp.sum(p * onehot, axis=0, keepdims=True)


def _fixup_body(plab_hbm, lab_hbm, fix_hbm, plab_v, lab_v, fix_v):
    cid = lax.axis_index("c")
    sid = lax.axis_index("s")
    wid = sid * NC + cid
    base = wid * CHUNK

    pltpu.sync_copy(plab_hbm.at[pl.ds(0, NBATCH), pl.ds(base, CHUNK)], plab_v)
    pltpu.sync_copy(lab_hbm.at[pl.ds(0, NBATCH), pl.ds(base, CHUNK)], lab_v)

    ones = jnp.ones((L,), jnp.float32)
    negones = jnp.full((L,), -1.0, jnp.float32)
    zvec = jnp.zeros((L,), jnp.float32)
    nbf = jnp.float32(NB)
    nbm1 = jnp.full((L,), NB - 1, jnp.int32)
    mirr = jnp.full((L,), NB2 - 1, jnp.int32)

    @plsc.parallel_loop(0, NUM_C * NB2 // L, unroll=8)
    def _zero(k):
        fix_v[pl.ds(k * L, L)] = zvec

    @plsc.parallel_loop(0, NBATCH * (CHUNK // L), unroll=8)
    def _vec(i):
        b = i // (CHUNK // L)
        j = i - b * (CHUNK // L)
        p = plab_v[b, pl.ds(j * L, L)]
        labv = lab_v[b, pl.ds(j * L, L)]
        row = labv * NB2
        bkt = jnp.minimum((p * nbf).astype(jnp.int32), nbm1)
        plsc.addupdate_scatter(fix_v, [row + bkt], negones)
        plsc.addupdate_scatter(fix_v, [row + (mirr - bkt)], ones)

    pltpu.sync_copy(fix_v, fix_hbm.at[wid])


def _stage2_tc_body(hist_ref, fix_ref, out_ref):
    accf = jnp.sum(fix_ref[...], axis=0)
    accbg = jnp.sum(hist_ref[...], axis=0)
    n_fg = accf[:, NB:]
    n_tot = accbg + accf[:, :NB] + n_fg
    row_i = lax.broadcasted_iota(jnp.int32, (NB, NB), 0)
    col_i = lax.broadcasted_iota(jnp.int32, (NB, NB), 1)
    m_tri = (row_i >= col_i).astype(jnp.float32)
    s_suf = jnp.dot(n_fg, m_tri, preferred_element_type=jnp.float32)
    k_suf = jnp.dot(n_tot, m_tri, preferred_element_type=jnp.float32)
    gts = s_suf[:, :1]
    union = gts + k_suf - s_suf
    ratio = (gts - s_suf) / jnp.maximum(union, 1.0)
    j = jnp.where(union > 0, 1.0 - ratio, 0.0)
    loss_c = jnp.sum(j, axis=1) * (1.0 / NB) - 0.5 / NB
    pres = jnp.where(gts[:, 0] > 0, 1.0, 0.0)
    loss = jnp.sum(loss_c * pres) / jnp.maximum(jnp.sum(pres), 1.0)
    out_ref[...] = jnp.broadcast_to(loss, (1, 1))


def _build_calls():
    mesh = plsc.VectorSubcoreMesh(
        core_axis_name="c", subcore_axis_name="s",
        num_cores=NC, num_subcores=NS)

    params = pltpu.CompilerParams(needs_layout_passes=False)

    stage1a = functools.partial(
        pl.kernel, _stage1a_body, mesh=mesh,
        compiler_params=params,
        out_type=jax.ShapeDtypeStruct((NW, NUM_C, NB), jnp.float32),
        scratch_types=[
            pltpu.VMEM((2, NBATCH, CHUNK), jnp.float32),
            pltpu.VMEM((NB,), jnp.float32),
            pltpu.SemaphoreType.DMA,
        ],
    )()

    plab_tc = pl.pallas_call(
        _plab_tc_body,
        grid=(NBATCH, PIX // (CR * 128)),
        in_specs=[
            pl.BlockSpec((1, NUM_C, CR, 128), lambda b, j: (b, 0, j, 0)),
            pl.BlockSpec((1, CR, 128), lambda b, j: (b, j, 0)),
        ],
        out_specs=pl.BlockSpec((1, CR, 128), lambda b, j: (b, j, 0)),
        out_shape=jax.ShapeDtypeStruct((NBATCH, PIX // 128, 128), jnp.float32),
    )

    fixup = functools.partial(
        pl.kernel, _fixup_body, mesh=mesh,
        compiler_params=params,
        out_type=jax.ShapeDtypeStruct((NW, NUM_C * NB2), jnp.float32),
        scratch_types=[
            pltpu.VMEM((NBATCH, CHUNK), jnp.float32),
            pltpu.VMEM((NBATCH, CHUNK), jnp.int32),
            pltpu.VMEM((NUM_C * NB2,), jnp.float32),
        ],
    )()

    stage2 = pl.pallas_call(
        _stage2_tc_body,
        out_shape=jax.ShapeDtypeStruct((1, 1), jnp.float32),
    )
    return stage1a, plab_tc, fixup, stage2


def kernel(probas, labels):
    b, c, h, w = probas.shape
    p3 = probas.reshape(b, c, h * w)
    lab2 = labels.reshape(b, h * w)
    stage1a, plab_tc, fixup, stage2 = _build_calls()
    p4 = p3.reshape(b, c, h * w // 128, 128)
    lab3 = lab2.reshape(b, h * w // 128, 128)
    plab = plab_tc(p4, lab3).reshape(b, h * w)
    hist = stage1a(p3)
    fix = fixup(plab, lab2)
    out = stage2(hist, fix.reshape(NW, NUM_C, NB2))
    return out[0, 0]


# speedup vs baseline: 1.3783x; 1.3783x over previous
import functools

import jax
import jax.numpy as jnp
from jax import lax
from jax.experimental import pallas as pl
from jax.experimental.pallas import tpu as pltpu
from jax.experimental.pallas import tpu_sc as plsc

NB = 1024
NB2 = 2 * NB
L = 16
NC = 2
NS = 16
NW = NC * NS
NUM_C = 21
PIX = 512 * 512
CHUNK = PIX // NW
NBATCH = 4
NR = 4


def _stage1_body(p_hbm, lab_hbm, out_hbm, lab_v, p_v, hist_v, in_sem):
    cid = lax.axis_index("c")
    sid = lax.axis_index("s")
    wid = sid * NC + cid
    base = wid * CHUNK

    pltpu.sync_copy(lab_hbm.at[pl.ds(0, NBATCH), pl.ds(base, CHUNK)], lab_v)

    ones = jnp.ones((L,), jnp.float32)
    zvec = jnp.zeros((L,), jnp.float32)
    nbf = jnp.float32(NB)
    nbm1 = jnp.full((L,), NB - 1, jnp.int32)
    mirr = jnp.full((L,), NB2 - 1, jnp.int32)
    laneoff = (jnp.arange(L, dtype=jnp.int32) % NR) * NB2

    def start_fetch(c, par):
        pltpu.make_async_copy(
            p_hbm.at[pl.ds(0, NBATCH), c, pl.ds(base, CHUNK)],
            p_v.at[par], in_sem).start()

    def wait_fetch(par):
        pltpu.make_async_copy(
            p_hbm.at[pl.ds(0, NBATCH), 0, pl.ds(base, CHUNK)],
            p_v.at[par], in_sem).wait()

    start_fetch(0, 0)

    def class_body(c, carry):
        par = lax.rem(c, 2)
        @plsc.parallel_loop(0, NR * NB2 // L, unroll=8)
        def _zero(k):
            hist_v[pl.ds(k * L, L)] = zvec

        wait_fetch(par)

        @pl.when(c + 1 < NUM_C)
        def _prefetch():
            start_fetch(c + 1, 1 - par)

        @plsc.parallel_loop(0, NBATCH * (CHUNK // L), unroll=8)
        def _vec(i):
            b = i // (CHUNK // L)
            j = i - b * (CHUNK // L)
            p = p_v[par, b, pl.ds(j * L, L)]
            labv = lab_v[b, pl.ds(j * L, L)]
            fg = labv == c
            bkt = jnp.minimum((p * nbf).astype(jnp.int32), nbm1)
            idx = jnp.where(fg, mirr - bkt, bkt)
            plsc.addupdate_scatter(hist_v, [idx + laneoff], ones)

        pltpu.sync_copy(hist_v, out_hbm.at[wid, c])
        return carry
    lax.fori_loop(0, NUM_C, class_body, 0)


def _stage23_tc_body(hist_ref, out_ref):
    acc = jnp.sum(hist_ref[...], axis=(0, 2))
    n_fg = acc[:, NB:]
    n_tot = acc[:, :NB] + n_fg
    row_i = lax.broadcasted_iota(jnp.int32, (NB, NB), 0)
    col_i = lax.broadcasted_iota(jnp.int32, (NB, NB), 1)
    m_tri = (row_i >= col_i).astype(jnp.float32)
    s_suf = jnp.dot(n_fg, m_tri, preferred_element_type=jnp.float32)
    k_suf = jnp.dot(n_tot, m_tri, preferred_element_type=jnp.float32)
    gts = s_suf[:, :1]
    union = gts + k_suf - s_suf
    ratio = (gts - s_suf) / jnp.maximum(union, 1.0)
    j = jnp.where(union > 0, 1.0 - ratio, 0.0)
    loss_c = jnp.sum(j, axis=1) * (1.0 / NB) - 0.5 / NB
    pres = jnp.where(gts[:, 0] > 0, 1.0, 0.0)
    loss = jnp.sum(loss_c * pres) / jnp.maximum(jnp.sum(pres), 1.0)
    out_ref[...] = jnp.broadcast_to(loss, (1, 1))


def _build_calls():
    mesh = plsc.VectorSubcoreMesh(
        core_axis_name="c", subcore_axis_name="s",
        num_cores=NC, num_subcores=NS)

    params = pltpu.CompilerParams(needs_layout_passes=False)

    stage1 = functools.partial(
        pl.kernel, _stage1_body, mesh=mesh,
        compiler_params=params,
        out_type=jax.ShapeDtypeStruct((NW, NUM_C, NR * NB2), jnp.float32),
        scratch_types=[
            pltpu.VMEM((NBATCH, CHUNK), jnp.int32),
            pltpu.VMEM((2, NBATCH, CHUNK), jnp.float32),
            pltpu.VMEM((NR * NB2,), jnp.float32),
            pltpu.SemaphoreType.DMA,
        ],
    )()

    stage23 = pl.pallas_call(
        _stage23_tc_body,
        out_shape=jax.ShapeDtypeStruct((1, 1), jnp.float32),
    )
    return stage1, stage23


def kernel(probas, labels):
    b, c, h, w = probas.shape
    p3 = probas.reshape(b, c, h * w)
    lab2 = labels.reshape(b, h * w)
    stage1, stage23 = _build_calls()
    hist = stage1(p3, lab2)
    out = stage23(hist.reshape(NW, NUM_C, NR, NB2))
    return out[0, 0]


# speedup vs baseline: 1.5541x; 1.1275x over previous
import functools

import jax
import jax.numpy as jnp
from jax import lax
from jax.experimental import pallas as pl
from jax.experimental.pallas import tpu as pltpu
from jax.experimental.pallas import tpu_sc as plsc

NB = 1024
NB2 = 2 * NB
L = 16
NC = 2
NS = 16
NW = NC * NS
NUM_C = 21
PIX = 512 * 512
CHUNK = PIX // NW
NBATCH = 4


def _stage1_body(p_hbm, lab_hbm, out_hbm, lab_v, p_v, hist_v, in_sem):
    cid = lax.axis_index("c")
    sid = lax.axis_index("s")
    wid = sid * NC + cid
    base = wid * CHUNK

    pltpu.sync_copy(lab_hbm.at[pl.ds(0, NBATCH), pl.ds(base, CHUNK)], lab_v)

    ones = jnp.ones((L,), jnp.float32)
    zvec = jnp.zeros((L,), jnp.float32)
    nbf = jnp.float32(NB)
    nbm1 = jnp.full((L,), NB - 1, jnp.int32)
    mirr = jnp.full((L,), NB2 - 1, jnp.int32)

    def start_fetch(c, par):
        pltpu.make_async_copy(
            p_hbm.at[pl.ds(0, NBATCH), c, pl.ds(base, CHUNK)],
            p_v.at[par], in_sem).start()

    def wait_fetch(par):
        pltpu.make_async_copy(
            p_hbm.at[pl.ds(0, NBATCH), 0, pl.ds(base, CHUNK)],
            p_v.at[par], in_sem).wait()

    start_fetch(0, 0)

    def class_body(c, carry):
        par = lax.rem(c, 2)
        def zero_chunk(k, _):
            hist_v[pl.ds(k * L, L)] = zvec
            return 0
        lax.fori_loop(0, NB2 // L, zero_chunk, 0)

        wait_fetch(par)

        @pl.when(c + 1 < NUM_C)
        def _prefetch():
            start_fetch(c + 1, 1 - par)

        @plsc.parallel_loop(0, NBATCH * (CHUNK // L), unroll=8)
        def _vec(i):
            b = i // (CHUNK // L)
            j = i - b * (CHUNK // L)
            p = p_v[par, b, pl.ds(j * L, L)]
            labv = lab_v[b, pl.ds(j * L, L)]
            fg = labv == c
            bkt = jnp.minimum((p * nbf).astype(jnp.int32), nbm1)
            idx = jnp.where(fg, mirr - bkt, bkt)
            plsc.addupdate_scatter(hist_v, [idx], ones)

        pltpu.sync_copy(hist_v, out_hbm.at[wid, c])
        return carry
    lax.fori_loop(0, NUM_C, class_body, 0)


def _stage23_tc_body(hist_ref, out_ref):
    acc = jnp.sum(hist_ref[...], axis=0)
    n_fg = acc[:, NB:]
    n_tot = acc[:, :NB] + n_fg
    row_i = lax.broadcasted_iota(jnp.int32, (NB, NB), 0)
    col_i = lax.broadcasted_iota(jnp.int32, (NB, NB), 1)
    m_tri = (row_i >= col_i).astype(jnp.float32)
    s_suf = jnp.dot(n_fg, m_tri, preferred_element_type=jnp.float32)
    k_suf = jnp.dot(n_tot, m_tri, preferred_element_type=jnp.float32)
    gts = s_suf[:, :1]
    union = gts + k_suf - s_suf
    ratio = (gts - s_suf) / jnp.maximum(union, 1.0)
    j = jnp.where(union > 0, 1.0 - ratio, 0.0)
    loss_c = jnp.sum(j, axis=1) * (1.0 / NB) - 0.5 / NB
    pres = jnp.where(gts[:, 0] > 0, 1.0, 0.0)
    loss = jnp.sum(loss_c * pres) / jnp.maximum(jnp.sum(pres), 1.0)
    out_ref[...] = jnp.broadcast_to(loss, (1, 1))


def _build_calls():
    mesh = plsc.VectorSubcoreMesh(
        core_axis_name="c", subcore_axis_name="s",
        num_cores=NC, num_subcores=NS)

    params = pltpu.CompilerParams(needs_layout_passes=False)

    stage1 = functools.partial(
        pl.kernel, _stage1_body, mesh=mesh,
        compiler_params=params,
        out_type=jax.ShapeDtypeStruct((NW, NUM_C, NB2), jnp.float32),
        scratch_types=[
            pltpu.VMEM((NBATCH, CHUNK), jnp.int32),
            pltpu.VMEM((2, NBATCH, CHUNK), jnp.float32),
            pltpu.VMEM((NB2,), jnp.float32),
            pltpu.SemaphoreType.DMA,
        ],
    )()

    stage23 = pl.pallas_call(
        _stage23_tc_body,
        out_shape=jax.ShapeDtypeStruct((1, 1), jnp.float32),
    )
    return stage1, stage23


def kernel(probas, labels):
    b, c, h, w = probas.shape
    p3 = probas.reshape(b, c, h * w)
    lab2 = labels.reshape(b, h * w)
    stage1, stage23 = _build_calls()
    hist = stage1(p3, lab2)
    out = stage23(hist)
    return out[0, 0]
